# Initial kernel scaffold; baseline (speedup 1.0000x reference)
import jax
import jax.numpy as jnp
from jax.experimental import pallas as pl


def kernel(x, embeddings):
    raise NotImplementedError("write your pallas kernel here")


# speedup vs baseline: 8.7407x; 8.7407x over previous
import functools

import jax
import jax.numpy as jnp
from jax import lax
from jax.experimental import pallas as pl
from jax.experimental.pallas import tpu as pltpu
from jax.experimental.pallas import tpu_sc as plsc

_VOCAB = 100000
_D = 64
_B = 4096
_L = 50
_NC, _NS = 2, 16
_NW = _NC * _NS
_BPW = _B // _NW
_RPC = 2
_CI = _RPC * _L
_NCHUNK = _BPW // _RPC
_NVR = _D // 16

_mesh = plsc.VectorSubcoreMesh(core_axis_name="c", subcore_axis_name="s")


@functools.partial(
    pl.kernel,
    mesh=_mesh,
    out_type=jax.ShapeDtypeStruct((_B, _D), jnp.float32),
    compiler_params=pltpu.CompilerParams(use_tc_tiling_on_sc=False),
    scratch_types=[
        pltpu.VMEM((_NCHUNK, _CI), jnp.int32),
        pltpu.VMEM((_CI, _D), jnp.float32),
        pltpu.VMEM((_CI, _D), jnp.float32),
        pltpu.VMEM((_BPW, _D), jnp.float32),
        pltpu.SemaphoreType.DMA,
        pltpu.SemaphoreType.DMA,
    ],
)
def _pooled_lookup(x_hbm, emb_hbm, out_hbm, idx_v, rows0, rows1, out_v,
                   sem0, sem1):
    wid = lax.axis_index("s") * _NC + lax.axis_index("c")
    pltpu.sync_copy(x_hbm.at[wid], idx_v)
    bufs = ((rows0, sem0), (rows1, sem1))

    def gather(c, buf, sem):
        return pltpu.make_async_copy(emb_hbm.at[idx_v.at[c]], buf, sem)

    gather(0, rows0, sem0).start()

    def outer(i, carry):
        c0 = i * 2
        for b in range(2):
            c = c0 + b
            nbuf, nsem = bufs[(b + 1) % 2]

            @pl.when(c + 1 < _NCHUNK)
            def _():
                gather(c + 1, nbuf, nsem).start()

            buf, sem = bufs[b]
            gather(c, buf, sem).wait()

            def red(l, accs):
                return tuple(
                    accs[r * _NVR + j] + buf[r * _L + l, pl.ds(j * 16, 16)]
                    for r in range(_RPC) for j in range(_NVR))

            accs = lax.fori_loop(
                0, _L, red,
                tuple(jnp.zeros((16,), jnp.float32)
                      for _ in range(_RPC * _NVR)))
            for r in range(_RPC):
                for j in range(_NVR):
                    out_v[c * _RPC + r, pl.ds(j * 16, 16)] = (
                        accs[r * _NVR + j] * (1.0 / _L))
        return carry

    lax.fori_loop(0, _NCHUNK // 2, outer, 0)
    pltpu.sync_copy(out_v, out_hbm.at[pl.ds(wid * _BPW, _BPW)])


def kernel(x, embeddings):
    xr = x.astype(jnp.int32).reshape(_NW, _NCHUNK, _CI)
    return _pooled_lookup(xr, embeddings)
